# baseline (speedup 1.0000x reference)
import functools

import jax
import jax.numpy as jnp
from jax import lax
from jax.experimental import pallas as pl
from jax.experimental.pallas import tpu as pltpu
from jax.experimental.pallas import tpu_sc as plsc

_D = 110
_DP = 112
_G = 80
_GPS = 2
_R = _G * _GPS


@functools.lru_cache(maxsize=None)
def _build_gather(N: int, J: int, V: int):
    info = plsc.get_sparse_core_info()
    NC, NS = info.num_cores, info.num_subcores
    NW = NC * NS
    B = N * J
    blocks_per_super = _R // J
    assert _R % J == 0 and B % (NW * _R) == 0
    b_per_w = B // NW
    n_rows = b_per_w // _G
    n_super = b_per_w // _R
    sc_words = _R * _D
    mesh = plsc.VectorSubcoreMesh(core_axis_name="c", subcore_axis_name="s")

    @functools.partial(
        pl.kernel,
        out_type=jax.ShapeDtypeStruct((N, J, _D), jnp.float32),
        mesh=mesh,
        scratch_types=[
            pltpu.VMEM((n_rows, _G), jnp.int32),
            pltpu.VMEM((2, _R, _DP), jnp.float32),
            pltpu.VMEM((2, blocks_per_super, J, _D), jnp.float32),
            pltpu.SemaphoreType.DMA,
            pltpu.SemaphoreType.DMA,
            pltpu.SemaphoreType.DMA,
            pltpu.SemaphoreType.DMA,
        ],
        compiler_params=pltpu.CompilerParams(use_tc_tiling_on_sc=False),
    )
    def gather(steps_hbm, table_hbm, out_hbm, idx_v, pad_v, cmp_v, g0, g1, w0, w1):
        sem_g = (g0, g1)
        sem_w = (w0, w1)
        wid = lax.axis_index("s") * NC + lax.axis_index("c")
        base_blk = wid * (b_per_w // J)
        pltpu.sync_copy(steps_hbm.at[pl.ds(wid * n_rows, n_rows)], idx_v)

        def gather_desc(t, b, i):
            return pltpu.make_async_copy(
                table_hbm.at[idx_v.at[t * _GPS + i]],
                pad_v.at[b].at[pl.ds(i * _G, _G)],
                sem_g[b],
            )

        def write_desc(t, b):
            return pltpu.make_async_copy(
                cmp_v.at[b],
                out_hbm.at[pl.ds(base_blk + t * blocks_per_super, blocks_per_super)],
                sem_w[b],
            )

        def compact(b):
            src = pad_v.at[b]
            dst = cmp_v.at[b]

            def blk_body(blk, carry2):
                def row_body(j, carry3):
                    row = src.at[blk * J + j]
                    drow = dst.at[blk, j]
                    for k in (0, 16, 32, 48, 64, 80, _D - 16):
                        drow[pl.ds(k, 16)] = row[pl.ds(k, 16)]
                    return carry3

                lax.fori_loop(0, J, row_body, 0)
                return carry2

            lax.fori_loop(0, blocks_per_super, blk_body, 0)

        for b in (0, 1):
            for i in range(_GPS):
                gather_desc(b, b, i).start()

        def pair_body(u, carry):
            for b in (0, 1):
                t = 2 * u + b
                for i in range(_GPS):
                    gather_desc(t, b, i).wait()

                @pl.when(t >= 2)
                def _():
                    write_desc(t - 2, b).wait()

                compact(b)
                write_desc(t, b).start()

                @pl.when(t + 2 < n_super)
                def _():
                    for i in range(_GPS):
                        gather_desc(t + 2, b, i).start()
            return carry

        lax.fori_loop(0, n_super // 2, pair_body, 0)
        for b in (0, 1):
            write_desc(n_super - 2 + b, b).wait()

    return gather


def kernel(steps, embedding):
    N, J = steps.shape
    V, D = embedding.shape
    flat = steps.reshape(N * J // _G, _G)
    emb_p = jnp.pad(embedding, ((0, 0), (0, _DP - D)))
    return _build_gather(N, J, V)(flat, emb_p)


# speedup vs baseline: 1.2902x; 1.2902x over previous
import functools

import jax
import jax.numpy as jnp
from jax import lax
from jax.experimental import pallas as pl
from jax.experimental.pallas import tpu as pltpu
from jax.experimental.pallas import tpu_sc as plsc

_D = 110
_DP = 128
_G = 80
_GPS = 2
_R = _G * _GPS


@functools.lru_cache(maxsize=None)
def _build_gather(N: int, J: int, V: int):
    info = plsc.get_sparse_core_info()
    NC, NS = info.num_cores, info.num_subcores
    NW = NC * NS
    B = N * J
    blocks_per_super = _R // J
    assert _R % J == 0 and B % (NW * _R) == 0
    b_per_w = B // NW
    n_rows = b_per_w // _G
    n_super = b_per_w // _R
    mesh = plsc.VectorSubcoreMesh(core_axis_name="c", subcore_axis_name="s")

    @functools.partial(
        pl.kernel,
        out_type=jax.ShapeDtypeStruct((N, J, _D), jnp.float32),
        mesh=mesh,
        scratch_types=[
            pltpu.VMEM((n_rows, _G), jnp.int32),
            pltpu.VMEM((2, _R, _DP), jnp.float32),
            pltpu.VMEM((2, blocks_per_super, J, _D), jnp.float32),
            pltpu.SemaphoreType.DMA,
            pltpu.SemaphoreType.DMA,
            pltpu.SemaphoreType.DMA,
            pltpu.SemaphoreType.DMA,
        ],
    )
    def gather(steps_hbm, table_hbm, out_hbm, idx_v, pad_v, cmp_v, g0, g1, w0, w1):
        sem_g = (g0, g1)
        sem_w = (w0, w1)
        wid = lax.axis_index("s") * NC + lax.axis_index("c")
        base_blk = wid * (b_per_w // J)
        pltpu.sync_copy(steps_hbm.at[pl.ds(wid * n_rows, n_rows)], idx_v)

        def gather_desc(t, b, i):
            return pltpu.make_async_copy(
                table_hbm.at[idx_v.at[t * _GPS + i]],
                pad_v.at[b].at[pl.ds(i * _G, _G)],
                sem_g[b],
            )

        def write_desc(t, b):
            return pltpu.make_async_copy(
                cmp_v.at[b],
                out_hbm.at[pl.ds(base_blk + t * blocks_per_super, blocks_per_super)],
                sem_w[b],
            )

        def compact(b):
            src = pad_v.at[b]
            dst = cmp_v.at[b]

            def blk_body(blk, carry2):
                for j in range(J):
                    row = src.at[blk * J + j]
                    drow = dst.at[blk, j]
                    for k in (0, 16, 32, 48, 64, 80, _D - 16):
                        drow[pl.ds(k, 16)] = row[pl.ds(k, 16)]
                return carry2

            lax.fori_loop(0, blocks_per_super, blk_body, 0)

        for b in (0, 1):
            for i in range(_GPS):
                gather_desc(b, b, i).start()

        def pair_body(u, carry):
            for b in (0, 1):
                t = 2 * u + b
                for i in range(_GPS):
                    gather_desc(t, b, i).wait()

                @pl.when(t >= 2)
                def _():
                    write_desc(t - 2, b).wait()

                compact(b)
                write_desc(t, b).start()

                @pl.when(t + 2 < n_super)
                def _():
                    for i in range(_GPS):
                        gather_desc(t + 2, b, i).start()
            return carry

        lax.fori_loop(0, n_super // 2, pair_body, 0)
        for b in (0, 1):
            write_desc(n_super - 2 + b, b).wait()

    return gather


def kernel(steps, embedding):
    N, J = steps.shape
    V, D = embedding.shape
    flat = steps.reshape(N * J // _G, _G)
    emb_p = jnp.pad(embedding, ((0, 0), (0, _DP - D)))
    return _build_gather(N, J, V)(flat, emb_p)
